# initial kernel scaffold (unmeasured)
import jax
import jax.numpy as jnp
from jax import lax
from jax.experimental import pallas as pl
from jax.experimental.pallas import tpu as pltpu

B, SQ, H, D = 8, 8, 16, 128
SKV_SHARD = 1024
N_SPLIT = 4
SKV_BLK = SKV_SHARD // N_SPLIT
SCALE = D ** -0.5


def _partial_body(r_ref, q_ref, k_ref, v_ref, o_ref, l_ref):
    q = q_ref[0, :, 0, :]
    k = k_ref[0, :, 0, :]
    v = v_ref[0, :, 0, :]
    s = lax.dot_general(
        q, k, (((1,), (1,)), ((), ())), preferred_element_type=jnp.float32
    )
    p = jnp.exp(s * SCALE)
    l = jnp.sum(p, axis=1, keepdims=True)
    o = lax.dot_general(
        p, v, (((1,), (0,)), ((), ())), preferred_element_type=jnp.float32
    )
    o_ref[0, 0, :, :] = o
    l_ref[:, :] = l


def _compute_partial(r, Q, K, V):
    return pl.pallas_call(
        _partial_body,
        grid_spec=pltpu.PrefetchScalarGridSpec(
            num_scalar_prefetch=1,
            grid=(B, H),
            in_specs=[
                pl.BlockSpec((1, SQ, 1, D), lambda b, h, r: (b, 0, h, 0)),
                pl.BlockSpec((1, SKV_BLK, 1, D), lambda b, h, r: (b, r[0], h, 0)),
                pl.BlockSpec((1, SKV_BLK, 1, D), lambda b, h, r: (b, r[0], h, 0)),
            ],
            out_specs=[
                pl.BlockSpec((1, 1, SQ, D), lambda b, h, r: (b, h, 0, 0)),
                pl.BlockSpec((SQ, 1), lambda b, h, r: (0, b * H + h)),
            ],
        ),
        out_shape=[
            jax.ShapeDtypeStruct((B, H, SQ, D), jnp.float32),
            jax.ShapeDtypeStruct((SQ, B * H), jnp.float32),
        ],
    )(r, Q, K, V)


def _allreduce_body(
    o_in, l_in, o_out, l_out, o_rcv, l_rcv, so_sem, ro_sem, sl_sem, rl_sem
):
    x = lax.axis_index("x")
    y = lax.axis_index("y")
    z = lax.axis_index("z")
    peers = [(1 - x, y, z), (x, 1 - y, z), (x, y, 1 - z)]

    barrier = pltpu.get_barrier_semaphore()
    for p in peers:
        pl.semaphore_signal(
            barrier, inc=1, device_id=p, device_id_type=pl.DeviceIdType.MESH
        )
    pl.semaphore_wait(barrier, 3)

    o_out[...] = o_in[...]
    l_out[...] = l_in[...]

    for s_idx, p in enumerate(peers):
        rdma_o = pltpu.make_async_remote_copy(
            src_ref=o_out,
            dst_ref=o_rcv.at[s_idx],
            send_sem=so_sem.at[s_idx],
            recv_sem=ro_sem.at[s_idx],
            device_id=p,
            device_id_type=pl.DeviceIdType.MESH,
        )
        rdma_l = pltpu.make_async_remote_copy(
            src_ref=l_out,
            dst_ref=l_rcv.at[s_idx],
            send_sem=sl_sem.at[s_idx],
            recv_sem=rl_sem.at[s_idx],
            device_id=p,
            device_id_type=pl.DeviceIdType.MESH,
        )
        rdma_o.start()
        rdma_l.start()
        rdma_o.wait()
        rdma_l.wait()
        o_out[...] += o_rcv[s_idx]
        l_out[...] += l_rcv[s_idx]


def _allreduce(o, l):
    return pl.pallas_call(
        _allreduce_body,
        out_shape=[
            jax.ShapeDtypeStruct(o.shape, o.dtype),
            jax.ShapeDtypeStruct(l.shape, l.dtype),
        ],
        in_specs=[
            pl.BlockSpec(memory_space=pltpu.VMEM),
            pl.BlockSpec(memory_space=pltpu.VMEM),
        ],
        out_specs=[
            pl.BlockSpec(memory_space=pltpu.VMEM),
            pl.BlockSpec(memory_space=pltpu.VMEM),
        ],
        scratch_shapes=[
            pltpu.VMEM((3,) + o.shape, jnp.float32),
            pltpu.VMEM((3,) + l.shape, jnp.float32),
            pltpu.SemaphoreType.DMA((3,)),
            pltpu.SemaphoreType.DMA((3,)),
            pltpu.SemaphoreType.DMA((3,)),
            pltpu.SemaphoreType.DMA((3,)),
        ],
        compiler_params=pltpu.CompilerParams(collective_id=0),
    )(o, l)


def kernel(Q, K, V):
    r = lax.axis_index("x") * 2 + lax.axis_index("y")
    r_arr = jnp.reshape(r, (1,)).astype(jnp.int32)
    o, l = _compute_partial(r_arr, Q, K, V)
    o, l = _allreduce(o, l)
    l_bhq = jnp.transpose(l.reshape(SQ, B, H), (1, 2, 0))[..., None]
    out = o / l_bhq
    return jnp.transpose(out, (0, 2, 1, 3))


# baseline (device time: 193624 ns/iter reference)
import jax
import jax.numpy as jnp
from jax import lax
from jax.experimental import pallas as pl
from jax.experimental.pallas import tpu as pltpu

B, SQ, H, D = 8, 8, 16, 128
SKV_SHARD = 1024
N_SPLIT = 4
SKV_BLK = SKV_SHARD // N_SPLIT
SCALE = D ** -0.5


def _partial_body(r_ref, q_ref, k_ref, v_ref, o_ref, l_ref):
    for h in range(H):
        q = q_ref[0, :, h * D:(h + 1) * D]
        k = k_ref[0, :, h * D:(h + 1) * D]
        v = v_ref[0, :, h * D:(h + 1) * D]
        s = lax.dot_general(
            q, k, (((1,), (1,)), ((), ())), preferred_element_type=jnp.float32
        )
        p = jnp.exp(s * SCALE)
        l = jnp.sum(p, axis=1, keepdims=True)
        o = lax.dot_general(
            p, v, (((1,), (0,)), ((), ())), preferred_element_type=jnp.float32
        )
        o_ref[0, :, h * D:(h + 1) * D] = o
        l_ref[0, :, h:h + 1] = l


def _compute_partial(r, Q, K, V):
    return pl.pallas_call(
        _partial_body,
        grid_spec=pltpu.PrefetchScalarGridSpec(
            num_scalar_prefetch=1,
            grid=(B,),
            in_specs=[
                pl.BlockSpec((1, SQ, H * D), lambda b, r: (b, 0, 0)),
                pl.BlockSpec((1, SKV_BLK, H * D), lambda b, r: (b, r[0], 0)),
                pl.BlockSpec((1, SKV_BLK, H * D), lambda b, r: (b, r[0], 0)),
            ],
            out_specs=[
                pl.BlockSpec((1, SQ, H * D), lambda b, r: (b, 0, 0)),
                pl.BlockSpec((1, SQ, H), lambda b, r: (b, 0, 0)),
            ],
        ),
        out_shape=[
            jax.ShapeDtypeStruct((B, SQ, H * D), jnp.float32),
            jax.ShapeDtypeStruct((B, SQ, H), jnp.float32),
        ],
    )(r, Q, K, V)


def _allreduce_body(
    o_in, l_in, o_out, l_out, o_rcv, l_rcv, so_sem, ro_sem, sl_sem, rl_sem
):
    x = lax.axis_index("x")
    y = lax.axis_index("y")
    z = lax.axis_index("z")
    peers = [(1 - x, y, z), (x, 1 - y, z), (x, y, 1 - z)]

    barrier = pltpu.get_barrier_semaphore()
    for p in peers:
        pl.semaphore_signal(
            barrier, inc=1, device_id=p, device_id_type=pl.DeviceIdType.MESH
        )
    pl.semaphore_wait(barrier, 3)

    o_out[...] = o_in[...]
    l_out[...] = l_in[...]

    for s_idx, p in enumerate(peers):
        rdma_o = pltpu.make_async_remote_copy(
            src_ref=o_out,
            dst_ref=o_rcv.at[s_idx],
            send_sem=so_sem.at[s_idx],
            recv_sem=ro_sem.at[s_idx],
            device_id=p,
            device_id_type=pl.DeviceIdType.MESH,
        )
        rdma_l = pltpu.make_async_remote_copy(
            src_ref=l_out,
            dst_ref=l_rcv.at[s_idx],
            send_sem=sl_sem.at[s_idx],
            recv_sem=rl_sem.at[s_idx],
            device_id=p,
            device_id_type=pl.DeviceIdType.MESH,
        )
        rdma_o.start()
        rdma_l.start()
        rdma_o.wait()
        rdma_l.wait()
        o_out[...] += o_rcv[s_idx]
        l_out[...] += l_rcv[s_idx]


def _allreduce(o, l):
    return pl.pallas_call(
        _allreduce_body,
        out_shape=[
            jax.ShapeDtypeStruct(o.shape, o.dtype),
            jax.ShapeDtypeStruct(l.shape, l.dtype),
        ],
        in_specs=[
            pl.BlockSpec(memory_space=pltpu.VMEM),
            pl.BlockSpec(memory_space=pltpu.VMEM),
        ],
        out_specs=[
            pl.BlockSpec(memory_space=pltpu.VMEM),
            pl.BlockSpec(memory_space=pltpu.VMEM),
        ],
        scratch_shapes=[
            pltpu.VMEM((3,) + o.shape, jnp.float32),
            pltpu.VMEM((3,) + l.shape, jnp.float32),
            pltpu.SemaphoreType.DMA((3,)),
            pltpu.SemaphoreType.DMA((3,)),
            pltpu.SemaphoreType.DMA((3,)),
            pltpu.SemaphoreType.DMA((3,)),
        ],
        compiler_params=pltpu.CompilerParams(collective_id=0),
    )(o, l)


def kernel(Q, K, V):
    r = lax.axis_index("x") * 2 + lax.axis_index("y")
    r_arr = jnp.reshape(r, (1,)).astype(jnp.int32)
    Q2 = Q.reshape(B, SQ, H * D)
    K2 = K.reshape(B, SKV_SHARD, H * D)
    V2 = V.reshape(B, SKV_SHARD, H * D)
    o, l = _compute_partial(r_arr, Q2, K2, V2)
    o, l = _allreduce(o, l)
    out = o.reshape(B, SQ, H, D) / l[..., None]
    return out


# device time: 67777 ns/iter; 2.8568x vs baseline; 2.8568x over previous
import jax
import jax.numpy as jnp
from jax import lax
from jax.experimental import pallas as pl
from jax.experimental.pallas import tpu as pltpu

B, SQ, H, D = 8, 8, 16, 128
SKV_SHARD = 1024
N_SPLIT = 4
SKV_BLK = SKV_SHARD // N_SPLIT
SCALE = D ** -0.5


def _partial_body(
    r_ref, q_any, k_any, v_any, o_ref, l_ref,
    qbuf, kbuf, vbuf, qsem, ksem, vsem,
):
    r = r_ref[0]
    b = pl.program_id(0)

    def copies(bb, slot):
        out = []
        for h in range(H):
            kv = pl.ds(r * SKV_BLK, SKV_BLK)
            out.append(pltpu.make_async_copy(
                k_any.at[bb, kv, h, :], kbuf.at[slot, h], ksem.at[slot, h]))
            out.append(pltpu.make_async_copy(
                v_any.at[bb, kv, h, :], vbuf.at[slot, h], vsem.at[slot, h]))
            out.append(pltpu.make_async_copy(
                q_any.at[bb, :, h, :], qbuf.at[slot, h], qsem.at[slot, h]))
        return out

    slot = lax.rem(b, 2)
    nslot = lax.rem(b + 1, 2)

    @pl.when(b == 0)
    def _():
        for c in copies(b, slot):
            c.start()

    @pl.when(b + 1 < B)
    def _():
        for c in copies(b + 1, nslot):
            c.start()

    l_cols = []
    for h in range(H):
        pltpu.make_async_copy(
            k_any.at[b, pl.ds(r * SKV_BLK, SKV_BLK), h, :],
            kbuf.at[slot, h], ksem.at[slot, h]).wait()
        pltpu.make_async_copy(
            v_any.at[b, pl.ds(r * SKV_BLK, SKV_BLK), h, :],
            vbuf.at[slot, h], vsem.at[slot, h]).wait()
        pltpu.make_async_copy(
            q_any.at[b, :, h, :], qbuf.at[slot, h], qsem.at[slot, h]).wait()
        q = qbuf[slot, h]
        k = kbuf[slot, h]
        v = vbuf[slot, h]
        s = lax.dot_general(
            q, k, (((1,), (1,)), ((), ())), preferred_element_type=jnp.float32
        )
        p = jnp.exp(s * SCALE)
        l_cols.append(jnp.sum(p, axis=1, keepdims=True))
        o = lax.dot_general(
            p, v, (((1,), (0,)), ((), ())), preferred_element_type=jnp.float32
        )
        o_ref[b, h, :, :] = o
    l_ref[b, :, :] = jnp.concatenate(l_cols, axis=1)


def _compute_partial(r, Q, K, V):
    return pl.pallas_call(
        _partial_body,
        grid_spec=pltpu.PrefetchScalarGridSpec(
            num_scalar_prefetch=1,
            grid=(B,),
            in_specs=[
                pl.BlockSpec(memory_space=pl.ANY),
                pl.BlockSpec(memory_space=pl.ANY),
                pl.BlockSpec(memory_space=pl.ANY),
            ],
            out_specs=[
                pl.BlockSpec(memory_space=pltpu.VMEM),
                pl.BlockSpec(memory_space=pltpu.VMEM),
            ],
            scratch_shapes=[
                pltpu.VMEM((2, H, SQ, D), jnp.float32),
                pltpu.VMEM((2, H, SKV_BLK, D), jnp.float32),
                pltpu.VMEM((2, H, SKV_BLK, D), jnp.float32),
                pltpu.SemaphoreType.DMA((2, H)),
                pltpu.SemaphoreType.DMA((2, H)),
                pltpu.SemaphoreType.DMA((2, H)),
            ],
        ),
        out_shape=[
            jax.ShapeDtypeStruct((B, H, SQ, D), jnp.float32),
            jax.ShapeDtypeStruct((B, SQ, H), jnp.float32),
        ],
    )(r, Q, K, V)


def _allreduce_body(
    o_in, l_in, o_out, l_out, o_rcv, l_rcv, so_sem, ro_sem, sl_sem, rl_sem
):
    x = lax.axis_index("x")
    y = lax.axis_index("y")
    z = lax.axis_index("z")
    peers = [(1 - x, y, z), (x, 1 - y, z), (x, y, 1 - z)]

    barrier = pltpu.get_barrier_semaphore()
    for p in peers:
        pl.semaphore_signal(
            barrier, inc=1, device_id=p, device_id_type=pl.DeviceIdType.MESH
        )
    pl.semaphore_wait(barrier, 3)

    o_out[...] = o_in[...]
    l_out[...] = l_in[...]

    for s_idx, p in enumerate(peers):
        rdma_o = pltpu.make_async_remote_copy(
            src_ref=o_out,
            dst_ref=o_rcv.at[s_idx],
            send_sem=so_sem.at[s_idx],
            recv_sem=ro_sem.at[s_idx],
            device_id=p,
            device_id_type=pl.DeviceIdType.MESH,
        )
        rdma_l = pltpu.make_async_remote_copy(
            src_ref=l_out,
            dst_ref=l_rcv.at[s_idx],
            send_sem=sl_sem.at[s_idx],
            recv_sem=rl_sem.at[s_idx],
            device_id=p,
            device_id_type=pl.DeviceIdType.MESH,
        )
        rdma_o.start()
        rdma_l.start()
        rdma_o.wait()
        rdma_l.wait()
        o_out[...] += o_rcv[s_idx]
        l_out[...] += l_rcv[s_idx]


def _allreduce(o, l):
    return pl.pallas_call(
        _allreduce_body,
        out_shape=[
            jax.ShapeDtypeStruct(o.shape, o.dtype),
            jax.ShapeDtypeStruct(l.shape, l.dtype),
        ],
        in_specs=[
            pl.BlockSpec(memory_space=pltpu.VMEM),
            pl.BlockSpec(memory_space=pltpu.VMEM),
        ],
        out_specs=[
            pl.BlockSpec(memory_space=pltpu.VMEM),
            pl.BlockSpec(memory_space=pltpu.VMEM),
        ],
        scratch_shapes=[
            pltpu.VMEM((3,) + o.shape, jnp.float32),
            pltpu.VMEM((3,) + l.shape, jnp.float32),
            pltpu.SemaphoreType.DMA((3,)),
            pltpu.SemaphoreType.DMA((3,)),
            pltpu.SemaphoreType.DMA((3,)),
            pltpu.SemaphoreType.DMA((3,)),
        ],
        compiler_params=pltpu.CompilerParams(collective_id=0),
    )(o, l)


def kernel(Q, K, V):
    r = lax.axis_index("x") * 2 + lax.axis_index("y")
    r_arr = jnp.reshape(r, (1,)).astype(jnp.int32)
    o, l = _compute_partial(r_arr, Q, K, V)
    o, l = _allreduce(o, l)
    out = o / jnp.transpose(l, (0, 2, 1))[..., None]
    return jnp.transpose(out, (0, 2, 1, 3))


# device time: 59135 ns/iter; 3.2743x vs baseline; 1.1461x over previous
import jax
import jax.numpy as jnp
from jax import lax
from jax.experimental import pallas as pl
from jax.experimental.pallas import tpu as pltpu

B, SQ, H, D = 8, 8, 16, 128
SKV_SHARD = 1024
N_SPLIT = 4
SKV_BLK = SKV_SHARD // N_SPLIT
SCALE = D ** -0.5


def _partial_body(
    r_ref, q_any, k_any, v_any, o_ref, l_ref,
    qbuf, kbuf, vbuf, qsem, ksem, vsem,
):
    r = r_ref[0]
    b = pl.program_id(0)

    def copies(bb, slot):
        out = []
        for h in range(H):
            kv = pl.ds(r * SKV_BLK, SKV_BLK)
            out.append(pltpu.make_async_copy(
                k_any.at[bb, kv, h, :], kbuf.at[slot, h], ksem.at[slot, h]))
            out.append(pltpu.make_async_copy(
                v_any.at[bb, kv, h, :], vbuf.at[slot, h], vsem.at[slot, h]))
            out.append(pltpu.make_async_copy(
                q_any.at[bb, :, h, :], qbuf.at[slot, h], qsem.at[slot, h]))
        return out

    slot = lax.rem(b, 2)
    nslot = lax.rem(b + 1, 2)

    @pl.when(b == 0)
    def _():
        for c in copies(b, slot):
            c.start()

    @pl.when(b + 1 < B)
    def _():
        for c in copies(b + 1, nslot):
            c.start()

    l_cols = []
    for h in range(H):
        pltpu.make_async_copy(
            k_any.at[b, pl.ds(r * SKV_BLK, SKV_BLK), h, :],
            kbuf.at[slot, h], ksem.at[slot, h]).wait()
        pltpu.make_async_copy(
            v_any.at[b, pl.ds(r * SKV_BLK, SKV_BLK), h, :],
            vbuf.at[slot, h], vsem.at[slot, h]).wait()
        pltpu.make_async_copy(
            q_any.at[b, :, h, :], qbuf.at[slot, h], qsem.at[slot, h]).wait()
        q = qbuf[slot, h]
        k = kbuf[slot, h]
        v = vbuf[slot, h]
        s = lax.dot_general(
            q, k, (((1,), (1,)), ((), ())), preferred_element_type=jnp.float32
        )
        p = jnp.exp(s * SCALE)
        l_cols.append(jnp.sum(p, axis=1, keepdims=True))
        o = lax.dot_general(
            p, v, (((1,), (0,)), ((), ())), preferred_element_type=jnp.float32
        )
        o_ref[b, h, :, :] = o
    l_ref[b, :, :] = jnp.concatenate(l_cols, axis=1)


def _compute_partial(r, Q, K, V):
    return pl.pallas_call(
        _partial_body,
        grid_spec=pltpu.PrefetchScalarGridSpec(
            num_scalar_prefetch=1,
            grid=(B,),
            in_specs=[
                pl.BlockSpec(memory_space=pl.ANY),
                pl.BlockSpec(memory_space=pl.ANY),
                pl.BlockSpec(memory_space=pl.ANY),
            ],
            out_specs=[
                pl.BlockSpec(memory_space=pltpu.VMEM),
                pl.BlockSpec(memory_space=pltpu.VMEM),
            ],
            scratch_shapes=[
                pltpu.VMEM((2, H, SQ, D), jnp.float32),
                pltpu.VMEM((2, H, SKV_BLK, D), jnp.float32),
                pltpu.VMEM((2, H, SKV_BLK, D), jnp.float32),
                pltpu.SemaphoreType.DMA((2, H)),
                pltpu.SemaphoreType.DMA((2, H)),
                pltpu.SemaphoreType.DMA((2, H)),
            ],
        ),
        out_shape=[
            jax.ShapeDtypeStruct((B, H, SQ, D), jnp.float32),
            jax.ShapeDtypeStruct((B, SQ, H), jnp.float32),
        ],
    )(r, Q, K, V)


def _allreduce_body(
    o_in, l_in, o_out, l_out, o_snd, o_rcv, l_rcv, so_sem, ro_sem, sl_sem, rl_sem
):
    x = lax.axis_index("x")
    y = lax.axis_index("y")
    z = lax.axis_index("z")
    peers = [(1 - x, y, z), (x, 1 - y, z), (x, y, 1 - z)]

    barrier = pltpu.get_barrier_semaphore()
    for p in peers:
        pl.semaphore_signal(
            barrier, inc=1, device_id=p, device_id_type=pl.DeviceIdType.MESH
        )
    pl.semaphore_wait(barrier, 3)

    o_srcs = [o_in, o_out, o_out]
    l_srcs = [l_in, l_out, l_out]
    for s_idx, p in enumerate(peers):
        o_snd[...] = o_srcs[s_idx][...].astype(jnp.bfloat16)
        rdma_o = pltpu.make_async_remote_copy(
            src_ref=o_snd,
            dst_ref=o_rcv.at[s_idx],
            send_sem=so_sem.at[s_idx],
            recv_sem=ro_sem.at[s_idx],
            device_id=p,
            device_id_type=pl.DeviceIdType.MESH,
        )
        rdma_l = pltpu.make_async_remote_copy(
            src_ref=l_srcs[s_idx],
            dst_ref=l_rcv.at[s_idx],
            send_sem=sl_sem.at[s_idx],
            recv_sem=rl_sem.at[s_idx],
            device_id=p,
            device_id_type=pl.DeviceIdType.MESH,
        )
        rdma_o.start()
        rdma_l.start()
        rdma_o.wait()
        rdma_l.wait()
        o_out[...] = o_srcs[s_idx][...] + o_rcv[s_idx].astype(jnp.float32)
        l_out[...] = l_srcs[s_idx][...] + l_rcv[s_idx]


def _allreduce(o, l):
    return pl.pallas_call(
        _allreduce_body,
        out_shape=[
            jax.ShapeDtypeStruct(o.shape, o.dtype),
            jax.ShapeDtypeStruct(l.shape, l.dtype),
        ],
        in_specs=[
            pl.BlockSpec(memory_space=pltpu.VMEM),
            pl.BlockSpec(memory_space=pltpu.VMEM),
        ],
        out_specs=[
            pl.BlockSpec(memory_space=pltpu.VMEM),
            pl.BlockSpec(memory_space=pltpu.VMEM),
        ],
        scratch_shapes=[
            pltpu.VMEM(o.shape, jnp.bfloat16),
            pltpu.VMEM((3,) + o.shape, jnp.bfloat16),
            pltpu.VMEM((3,) + l.shape, jnp.float32),
            pltpu.SemaphoreType.DMA((3,)),
            pltpu.SemaphoreType.DMA((3,)),
            pltpu.SemaphoreType.DMA((3,)),
            pltpu.SemaphoreType.DMA((3,)),
        ],
        compiler_params=pltpu.CompilerParams(collective_id=0),
    )(o, l)


def kernel(Q, K, V):
    r = lax.axis_index("x") * 2 + lax.axis_index("y")
    r_arr = jnp.reshape(r, (1,)).astype(jnp.int32)
    o, l = _compute_partial(r_arr, Q, K, V)
    o, l = _allreduce(o, l)
    out = o / jnp.transpose(l, (0, 2, 1))[..., None]
    return jnp.transpose(out, (0, 2, 1, 3))


# device time: 44175 ns/iter; 4.3831x vs baseline; 1.3387x over previous
import jax
import jax.numpy as jnp
from jax import lax
from jax.experimental import pallas as pl
from jax.experimental.pallas import tpu as pltpu

B, SQ, H, D = 8, 8, 16, 128
SKV_SHARD = 1024
N_SPLIT = 4
SKV_BLK = SKV_SHARD // N_SPLIT
SCALE = D ** -0.5


def _partial_body(r_ref, q_ref, k_ref, v_ref, o_ref, l_ref):
    l_cols = []
    for h in range(H):
        q = q_ref[0, :, h, :]
        k = k_ref[0, :, h, :]
        v = v_ref[0, :, h, :]
        s = lax.dot_general(
            q, k, (((1,), (1,)), ((), ())), preferred_element_type=jnp.float32
        )
        p = jnp.exp(s * SCALE)
        l_cols.append(jnp.sum(p, axis=1, keepdims=True))
        o = lax.dot_general(
            p, v, (((1,), (0,)), ((), ())), preferred_element_type=jnp.float32
        )
        o_ref[0, h, :, :] = o
    l_ref[0, :, :] = jnp.concatenate(l_cols, axis=1)


def _compute_partial(r, Q, K, V):
    return pl.pallas_call(
        _partial_body,
        grid_spec=pltpu.PrefetchScalarGridSpec(
            num_scalar_prefetch=1,
            grid=(B,),
            in_specs=[
                pl.BlockSpec((1, SQ, H, D), lambda b, r: (b, 0, 0, 0)),
                pl.BlockSpec((1, SKV_BLK, H, D), lambda b, r: (b, r[0], 0, 0)),
                pl.BlockSpec((1, SKV_BLK, H, D), lambda b, r: (b, r[0], 0, 0)),
            ],
            out_specs=[
                pl.BlockSpec((1, H, SQ, D), lambda b, r: (b, 0, 0, 0)),
                pl.BlockSpec((1, SQ, H), lambda b, r: (b, 0, 0)),
            ],
        ),
        out_shape=[
            jax.ShapeDtypeStruct((B, H, SQ, D), jnp.float32),
            jax.ShapeDtypeStruct((B, SQ, H), jnp.float32),
        ],
    )(r, Q, K, V)


def _allreduce_body(
    o_in, l_in, o_out, l_out, o_snd, o_rcv, l_rcv, so_sem, ro_sem, sl_sem, rl_sem
):
    x = lax.axis_index("x")
    y = lax.axis_index("y")
    z = lax.axis_index("z")
    peers = [(1 - x, y, z), (x, 1 - y, z), (x, y, 1 - z)]

    barrier = pltpu.get_barrier_semaphore()
    for p in peers:
        pl.semaphore_signal(
            barrier, inc=1, device_id=p, device_id_type=pl.DeviceIdType.MESH
        )
    pl.semaphore_wait(barrier, 3)

    o_srcs = [o_in, o_out, o_out]
    l_srcs = [l_in, l_out, l_out]
    for s_idx, p in enumerate(peers):
        o_snd[...] = o_srcs[s_idx][...].astype(jnp.bfloat16)
        rdma_o = pltpu.make_async_remote_copy(
            src_ref=o_snd,
            dst_ref=o_rcv.at[s_idx],
            send_sem=so_sem.at[s_idx],
            recv_sem=ro_sem.at[s_idx],
            device_id=p,
            device_id_type=pl.DeviceIdType.MESH,
        )
        rdma_l = pltpu.make_async_remote_copy(
            src_ref=l_srcs[s_idx],
            dst_ref=l_rcv.at[s_idx],
            send_sem=sl_sem.at[s_idx],
            recv_sem=rl_sem.at[s_idx],
            device_id=p,
            device_id_type=pl.DeviceIdType.MESH,
        )
        rdma_o.start()
        rdma_l.start()
        rdma_o.wait()
        rdma_l.wait()
        o_out[...] = o_srcs[s_idx][...] + o_rcv[s_idx].astype(jnp.float32)
        l_out[...] = l_srcs[s_idx][...] + l_rcv[s_idx]


def _allreduce(o, l):
    return pl.pallas_call(
        _allreduce_body,
        out_shape=[
            jax.ShapeDtypeStruct(o.shape, o.dtype),
            jax.ShapeDtypeStruct(l.shape, l.dtype),
        ],
        in_specs=[
            pl.BlockSpec(memory_space=pltpu.VMEM),
            pl.BlockSpec(memory_space=pltpu.VMEM),
        ],
        out_specs=[
            pl.BlockSpec(memory_space=pltpu.VMEM),
            pl.BlockSpec(memory_space=pltpu.VMEM),
        ],
        scratch_shapes=[
            pltpu.VMEM(o.shape, jnp.bfloat16),
            pltpu.VMEM((3,) + o.shape, jnp.bfloat16),
            pltpu.VMEM((3,) + l.shape, jnp.float32),
            pltpu.SemaphoreType.DMA((3,)),
            pltpu.SemaphoreType.DMA((3,)),
            pltpu.SemaphoreType.DMA((3,)),
            pltpu.SemaphoreType.DMA((3,)),
        ],
        compiler_params=pltpu.CompilerParams(collective_id=0),
    )(o, l)


def kernel(Q, K, V):
    r = lax.axis_index("x") * 2 + lax.axis_index("y")
    r_arr = jnp.reshape(r, (1,)).astype(jnp.int32)
    o, l = _compute_partial(r_arr, Q, K, V)
    o, l = _allreduce(o, l)
    out = o / jnp.transpose(l, (0, 2, 1))[..., None]
    return jnp.transpose(out, (0, 2, 1, 3))
